# baseline (speedup 1.0000x reference)
import jax
import jax.numpy as jnp
from jax.experimental import pallas as pl
from jax.experimental.pallas import tpu as pltpu

B, N, C = 16, 8732, 85
NCLS = C - 4
EPS = 1e-7
BISECT_ITERS = 32


def _stage_a(pred_ref, gt_ref, ent_ref, npos_ref, possum_ref, hubsum_ref):
    p = pred_ref[0]
    g = gt_ref[0]
    col = jax.lax.broadcasted_iota(jnp.int32, (N, C), 1)
    q = jnp.where(g > 0.5, p, 1.0 - p)
    bce = -jnp.log(jnp.clip(q, EPS, 1.0 - EPS))
    ent = jnp.sum(jnp.where(col < NCLS, bce, 0.0), axis=1, keepdims=True)
    pos = g[:, 0:1] < 0.5
    ent_neg = jnp.where(pos, 0.0, ent)
    ent_ref[0] = ent_neg.reshape(1, N)
    posf = pos.astype(jnp.float32)
    npos_ref[...] = jnp.full((1, 1, 1), jnp.sum(posf))
    possum_ref[...] = jnp.full((1, 1, 1), jnp.sum(jnp.where(pos, ent, 0.0)))
    d = p - g
    ad = jnp.abs(d)
    hub = jnp.where(ad < 1.0, 0.5 * d * d, ad - 0.5)
    hubm = jnp.where((col >= NCLS) & pos, hub, 0.0)
    hubsum_ref[...] = jnp.full((1, 1, 1), jnp.sum(hubm))


def _stage_b(ent_ref, npos_ref, possum_ref, hubsum_ref,
             all_ref, conf_ref, loc_ref):
    e = ent_ref[...]
    npos = npos_ref[...]
    k = 3.0 * npos

    lo = jnp.zeros((B, 1), jnp.float32)
    hi = jnp.max(e, axis=1, keepdims=True)

    def body(_, carry):
        lo, hi = carry
        mid = 0.5 * (lo + hi)
        cnt = jnp.sum((e > mid).astype(jnp.float32), axis=1, keepdims=True)
        ge = cnt >= k
        return jnp.where(ge, mid, lo), jnp.where(ge, hi, mid)

    lo, hi = jax.lax.fori_loop(0, BISECT_ITERS, body, (lo, hi))
    t = 0.5 * (lo + hi)
    above = e > t
    cnt_t = jnp.sum(above.astype(jnp.float32), axis=1, keepdims=True)
    s_above = jnp.sum(jnp.where(above, e, 0.0), axis=1, keepdims=True)
    neg_row = s_above + (k - cnt_t) * t

    neg_total = jnp.sum(neg_row)
    npos_total = jnp.sum(npos)
    pos_total = jnp.sum(possum_ref[...])
    hub_total = jnp.sum(hubsum_ref[...])

    loss_conf = (pos_total + neg_total) / npos_total
    loss_loc = hub_total / (npos_total * 4.0)
    loss_all = loss_conf + loss_loc
    all_ref[...] = jnp.full((1, 1), loss_all)
    conf_ref[...] = jnp.full((1, 1), loss_conf)
    loc_ref[...] = jnp.full((1, 1), loss_loc)


@jax.jit
def kernel(pred, gt):
    ent, npos, possum, hubsum = pl.pallas_call(
        _stage_a,
        grid=(B,),
        in_specs=[
            pl.BlockSpec((1, N, C), lambda i: (i, 0, 0)),
            pl.BlockSpec((1, N, C), lambda i: (i, 0, 0)),
        ],
        out_specs=[
            pl.BlockSpec((1, 1, N), lambda i: (i, 0, 0)),
            pl.BlockSpec((1, 1, 1), lambda i: (i, 0, 0)),
            pl.BlockSpec((1, 1, 1), lambda i: (i, 0, 0)),
            pl.BlockSpec((1, 1, 1), lambda i: (i, 0, 0)),
        ],
        out_shape=[
            jax.ShapeDtypeStruct((B, 1, N), jnp.float32),
            jax.ShapeDtypeStruct((B, 1, 1), jnp.float32),
            jax.ShapeDtypeStruct((B, 1, 1), jnp.float32),
            jax.ShapeDtypeStruct((B, 1, 1), jnp.float32),
        ],
    )(pred, gt)

    ent2d = ent.reshape(B, N)
    npos = npos.reshape(B, 1)
    possum = possum.reshape(B, 1)
    hubsum = hubsum.reshape(B, 1)
    loss_all, loss_conf, loss_loc = pl.pallas_call(
        _stage_b,
        out_shape=[
            jax.ShapeDtypeStruct((1, 1), jnp.float32),
            jax.ShapeDtypeStruct((1, 1), jnp.float32),
            jax.ShapeDtypeStruct((1, 1), jnp.float32),
        ],
    )(ent2d, npos, possum, hubsum)

    return (loss_all.reshape(()), loss_conf.reshape(()), loss_loc.reshape(()))


# speedup vs baseline: 5.6145x; 5.6145x over previous
import jax
import jax.numpy as jnp
from jax.experimental import pallas as pl
from jax.experimental.pallas import tpu as pltpu

B, N, C = 16, 8732, 85
NCLS = C - 4
EPS = 1e-7
BISECT_ITERS = 32
NC_BLK = 1024
NSTEPS = -(-N // NC_BLK)


def _stage_a(pt_ref, gt_ref, ent_ref, npos_ref, possum_ref, hubsum_ref):
    j = pl.program_id(0)
    p = pt_ref[...]
    g = gt_ref[...]
    lane = jax.lax.broadcasted_iota(jnp.int32, (B, NC_BLK), 1)
    valid = (lane + j * NC_BLK) < N

    pc = p[:NCLS]
    gc = g[:NCLS]
    q = jnp.where(gc > 0.5, pc, 1.0 - pc)
    bce = -jnp.log(jnp.clip(q, EPS, 1.0 - EPS))
    ent = jnp.sum(bce, axis=0)

    pos = g[0] < 0.5
    posv = pos & valid
    ent_ref[...] = jnp.where(pos, 0.0, ent)

    npos_j = jnp.sum(jnp.where(posv, 1.0, 0.0), axis=1, keepdims=True)
    possum_j = jnp.sum(jnp.where(posv, ent, 0.0), axis=1, keepdims=True)

    d = p[NCLS:] - g[NCLS:]
    ad = jnp.abs(d)
    hub = jnp.where(ad < 1.0, 0.5 * d * d, ad - 0.5)
    hubrow = jnp.sum(hub, axis=0)
    hubsum_j = jnp.sum(jnp.where(posv, hubrow, 0.0), axis=1, keepdims=True)

    @pl.when(j == 0)
    def _():
        npos_ref[...] = npos_j
        possum_ref[...] = possum_j
        hubsum_ref[...] = hubsum_j

    @pl.when(j > 0)
    def _():
        npos_ref[...] = npos_ref[...] + npos_j
        possum_ref[...] = possum_ref[...] + possum_j
        hubsum_ref[...] = hubsum_ref[...] + hubsum_j


def _stage_b(ent_ref, npos_ref, possum_ref, hubsum_ref,
             all_ref, conf_ref, loc_ref):
    e = ent_ref[...]
    npos = npos_ref[...]
    k = 3.0 * npos

    lo = jnp.zeros((B, 1), jnp.float32)
    hi = jnp.max(e, axis=1, keepdims=True)

    def body(_, carry):
        lo, hi = carry
        mid = 0.5 * (lo + hi)
        cnt = jnp.sum((e > mid).astype(jnp.float32), axis=1, keepdims=True)
        ge = cnt >= k
        return jnp.where(ge, mid, lo), jnp.where(ge, hi, mid)

    lo, hi = jax.lax.fori_loop(0, BISECT_ITERS, body, (lo, hi))
    t = 0.5 * (lo + hi)
    above = e > t
    cnt_t = jnp.sum(above.astype(jnp.float32), axis=1, keepdims=True)
    s_above = jnp.sum(jnp.where(above, e, 0.0), axis=1, keepdims=True)
    neg_row = s_above + (k - cnt_t) * t

    neg_total = jnp.sum(neg_row)
    npos_total = jnp.sum(npos)
    pos_total = jnp.sum(possum_ref[...])
    hub_total = jnp.sum(hubsum_ref[...])

    loss_conf = (pos_total + neg_total) / npos_total
    loss_loc = hub_total / (npos_total * 4.0)
    loss_all = loss_conf + loss_loc
    all_ref[...] = jnp.full((1, 1), loss_all)
    conf_ref[...] = jnp.full((1, 1), loss_conf)
    loc_ref[...] = jnp.full((1, 1), loss_loc)


@jax.jit
def kernel(pred, gt):
    pred_t = jnp.transpose(pred, (2, 0, 1))
    gt_t = jnp.transpose(gt, (2, 0, 1))

    ent, npos, possum, hubsum = pl.pallas_call(
        _stage_a,
        grid=(NSTEPS,),
        in_specs=[
            pl.BlockSpec((C, B, NC_BLK), lambda j: (0, 0, j)),
            pl.BlockSpec((C, B, NC_BLK), lambda j: (0, 0, j)),
        ],
        out_specs=[
            pl.BlockSpec((B, NC_BLK), lambda j: (0, j)),
            pl.BlockSpec((B, 1), lambda j: (0, 0)),
            pl.BlockSpec((B, 1), lambda j: (0, 0)),
            pl.BlockSpec((B, 1), lambda j: (0, 0)),
        ],
        out_shape=[
            jax.ShapeDtypeStruct((B, N), jnp.float32),
            jax.ShapeDtypeStruct((B, 1), jnp.float32),
            jax.ShapeDtypeStruct((B, 1), jnp.float32),
            jax.ShapeDtypeStruct((B, 1), jnp.float32),
        ],
    )(pred_t, gt_t)

    loss_all, loss_conf, loss_loc = pl.pallas_call(
        _stage_b,
        out_shape=[
            jax.ShapeDtypeStruct((1, 1), jnp.float32),
            jax.ShapeDtypeStruct((1, 1), jnp.float32),
            jax.ShapeDtypeStruct((1, 1), jnp.float32),
        ],
    )(ent, npos, possum, hubsum)

    return (loss_all.reshape(()), loss_conf.reshape(()), loss_loc.reshape(()))
